# baseline (speedup 1.0000x reference)
import functools

import jax
import jax.numpy as jnp
from jax import lax
from jax.experimental import pallas as pl
from jax.experimental.pallas import tpu as pltpu
from jax.experimental.pallas import tpu_sc as plsc

VOCAB = 100000
D = 128
BATCH = 1024
SEQ = 200
B = BATCH * SEQ

NC = 2
NS = 16
NW = NC * NS
B_PER_W = B // NW
CHUNK = 128
N_CHUNK = B_PER_W // CHUNK
NBUF = 6
LOOKAHEAD = 3

_mesh = plsc.VectorSubcoreMesh(core_axis_name="c", subcore_axis_name="s")


@functools.partial(
    pl.kernel,
    mesh=_mesh,
    out_type=jax.ShapeDtypeStruct((B, D), jnp.float32),
    scratch_types=[
        pltpu.VMEM((N_CHUNK, CHUNK), jnp.int32),
        pltpu.VMEM((NBUF, CHUNK, D), jnp.float32),
        pltpu.SemaphoreType.DMA((NBUF,)),
        pltpu.SemaphoreType.DMA((NBUF,)),
    ],
)
def _gather_kernel(idx_hbm, table_hbm, out_hbm, idx_v, rows_v, gsems, ssems):
    wid = lax.axis_index("s") * NC + lax.axis_index("c")
    base = wid * B_PER_W
    pltpu.sync_copy(idx_hbm.at[wid], idx_v)

    def gather(j, p):
        return pltpu.make_async_copy(
            table_hbm.at[idx_v.at[j]], rows_v.at[p], gsems.at[p])

    def store(j, p):
        return pltpu.make_async_copy(
            rows_v.at[p], out_hbm.at[pl.ds(base + j * CHUNK, CHUNK)],
            ssems.at[p])

    for k in range(LOOKAHEAD):
        gather(k, k).start()

    def step(j, carry):
        p = j % NBUF
        gather(j, p).wait()
        store(j, p).start()

        nxt = j + LOOKAHEAD

        @pl.when(nxt < N_CHUNK)
        def _():
            q = nxt % NBUF

            @pl.when(nxt >= NBUF)
            def _():
                store(nxt - NBUF, q).wait()

            gather(nxt, q).start()

        return carry

    lax.fori_loop(0, N_CHUNK, step, 0)

    for t in range(N_CHUNK - NBUF, N_CHUNK):
        store(t, t % NBUF).wait()


def kernel(inputs, wte):
    idx = inputs.reshape(NW, N_CHUNK, CHUNK).astype(jnp.int32)
    out = _gather_kernel(idx, wte)
    return out.reshape(BATCH, SEQ, D)


# speedup vs baseline: 1.0039x; 1.0039x over previous
import functools

import jax
import jax.numpy as jnp
from jax import lax
from jax.experimental import pallas as pl
from jax.experimental.pallas import tpu as pltpu
from jax.experimental.pallas import tpu_sc as plsc

VOCAB = 100000
D = 128
BATCH = 1024
SEQ = 200
B = BATCH * SEQ

NC = 2
NS = 16
NW = NC * NS
B_PER_W = B // NW
CHUNK = 64
N_CHUNK = B_PER_W // CHUNK
NBUF = 12
LOOKAHEAD = 6

_mesh = plsc.VectorSubcoreMesh(core_axis_name="c", subcore_axis_name="s")


@functools.partial(
    pl.kernel,
    mesh=_mesh,
    out_type=jax.ShapeDtypeStruct((B, D), jnp.float32),
    scratch_types=[
        pltpu.VMEM((N_CHUNK, CHUNK), jnp.int32),
        pltpu.VMEM((NBUF, CHUNK, D), jnp.float32),
        pltpu.SemaphoreType.DMA((NBUF,)),
        pltpu.SemaphoreType.DMA((NBUF,)),
    ],
)
def _gather_kernel(idx_hbm, table_hbm, out_hbm, idx_v, rows_v, gsems, ssems):
    wid = lax.axis_index("s") * NC + lax.axis_index("c")
    base = wid * B_PER_W
    pltpu.sync_copy(idx_hbm.at[wid], idx_v)

    def gather(j, p):
        return pltpu.make_async_copy(
            table_hbm.at[idx_v.at[j]], rows_v.at[p], gsems.at[p])

    def store(j, p):
        return pltpu.make_async_copy(
            rows_v.at[p], out_hbm.at[pl.ds(base + j * CHUNK, CHUNK)],
            ssems.at[p])

    for k in range(LOOKAHEAD):
        gather(k, k).start()

    def step(j, carry):
        p = j % NBUF
        gather(j, p).wait()
        store(j, p).start()

        nxt = j + LOOKAHEAD

        @pl.when(nxt < N_CHUNK)
        def _():
            q = nxt % NBUF

            @pl.when(nxt >= NBUF)
            def _():
                store(nxt - NBUF, q).wait()

            gather(nxt, q).start()

        return carry

    lax.fori_loop(0, N_CHUNK, step, 0)

    for t in range(N_CHUNK - NBUF, N_CHUNK):
        store(t, t % NBUF).wait()


def kernel(inputs, wte):
    idx = inputs.reshape(NW, N_CHUNK, CHUNK).astype(jnp.int32)
    out = _gather_kernel(idx, wte)
    return out.reshape(BATCH, SEQ, D)
